# baseline (speedup 1.0000x reference)
import jax
import jax.numpy as jnp
from jax import lax
from jax.experimental import pallas as pl
from jax.experimental.pallas import tpu as pltpu
from jax.experimental.pallas import tpu_sc as plsc

N = 10000
E = 320000
NC = 2
NS = 16
NW = NC * NS
NP = 10240
RPT = NP // NS
CH = 128
EPW = NP
EPAD = NW * EPW
NCHUNK = EPW // CH
NBLK = 10
IB = NCHUNK // NBLK


def _sc_aggregate(table, src4, dst4, zrows, with_deg, zdeg=None, ones=None):
    D = table.shape[1]
    mesh = plsc.VectorSubcoreMesh(
        core_axis_name="c", subcore_axis_name="s", num_cores=NC, num_subcores=NS
    )
    out_type = [jax.ShapeDtypeStruct((NC * NP, D), jnp.float32)]
    if with_deg:
        out_type.append(jax.ShapeDtypeStruct((NC * NP,), jnp.float32))
    scratch = {
        "acc_sh": pltpu.VMEM_SHARED((NP, D), jnp.float32),
        "src_v": pltpu.VMEM((IB, CH), jnp.int32),
        "dst_v": pltpu.VMEM((IB, CH), jnp.int32),
        "dsem": pltpu.SemaphoreType.DMA,
    }
    RB = 2 if D > 64 else 4
    for r in range(RB):
        scratch[f"rows_v{r}"] = pltpu.VMEM((CH, D), jnp.float32)
        scratch[f"gsem{r}"] = pltpu.SemaphoreType.DMA
        scratch[f"ssem{r}"] = pltpu.SemaphoreType.DMA
    if with_deg:
        scratch["deg_sh"] = pltpu.VMEM_SHARED((NP, 16), jnp.float32)
        scratch["ones_v"] = pltpu.VMEM((CH, 16), jnp.float32)
        scratch["dbounce_v"] = pltpu.VMEM((64, 16), jnp.float32)
        scratch["degf_v"] = pltpu.VMEM((RPT,), jnp.float32)

    def body(*refs, acc_sh, src_v, dst_v, dsem, deg_sh=None, ones_v=None,
             dbounce_v=None, degf_v=None, **ring):
        if with_deg:
            (table_h, src_h, dst_h, zrows_h, zdeg_h, ones_h,
             acc_out, deg_out) = refs
        else:
            (table_h, src_h, dst_h, zrows_h, acc_out) = refs
        c = lax.axis_index("c")
        s = lax.axis_index("s")
        wid = c * NS + s

        pltpu.sync_copy(zrows_h, acc_sh.at[pl.ds(s * RPT, RPT)])
        if with_deg:
            pltpu.sync_copy(zdeg_h, deg_sh.at[pl.ds(s * RPT, RPT)])
            pltpu.sync_copy(ones_h, ones_v)
        plsc.subcore_barrier()

        rows = [ring[f"rows_v{r}"] for r in range(RB)]
        gsems = [ring[f"gsem{r}"] for r in range(RB)]
        ssems = [ring[f"ssem{r}"] for r in range(RB)]

        def blk_step(b, carry):
            pltpu.sync_copy(src_h.at[wid, b], src_v)
            pltpu.sync_copy(dst_h.at[wid, b], dst_v)
            gd = [None] * IB
            sd = [None] * IB
            gd[0] = pltpu.async_copy(table_h.at[src_v.at[0]], rows[0], gsems[0])
            deg_d = []
            for j in range(IB):
                if j + 1 < IB:
                    if j + 1 >= RB:
                        sd[j + 1 - RB].wait()
                    gd[j + 1] = pltpu.async_copy(
                        table_h.at[src_v.at[j + 1]], rows[(j + 1) % RB],
                        gsems[(j + 1) % RB])
                gd[j].wait()
                sd[j] = pltpu.async_copy(rows[j % RB],
                                         acc_sh.at[dst_v.at[j]],
                                         ssems[j % RB], add=True)
                if with_deg:
                    deg_d.append(pltpu.async_copy(
                        ones_v, deg_sh.at[dst_v.at[j]], dsem, add=True))
            for t in range(max(0, IB - RB), IB):
                sd[t].wait()
            for dd in deg_d:
                dd.wait()
            return carry

        lax.fori_loop(0, NBLK, blk_step, 0)
        plsc.subcore_barrier()

        pltpu.sync_copy(acc_sh.at[pl.ds(s * RPT, RPT)],
                        acc_out.at[pl.ds(c * NP + s * RPT, RPT)])
        if with_deg:
            iota16 = lax.iota(jnp.int32, 16)

            def tr_step(p, carry):
                pltpu.sync_copy(deg_sh.at[pl.ds(s * RPT + p * 64, 64)],
                                dbounce_v)
                for t in range(4):
                    vec = jnp.zeros((16,), jnp.float32)
                    for k in range(16):
                        vec = jnp.where(iota16 == k,
                                        dbounce_v[16 * t + k, :], vec)
                    degf_v[pl.ds(p * 64 + t * 16, 16)] = vec
                return carry

            lax.fori_loop(0, RPT // 64, tr_step, 0)
            pltpu.sync_copy(degf_v, deg_out.at[pl.ds(c * NP + s * RPT, RPT)])

    run = pl.kernel(body, out_type=out_type, mesh=mesh, scratch_types=scratch,
                    compiler_params=pltpu.CompilerParams(
                        use_tc_tiling_on_sc=False))
    if with_deg:
        return run(table, src4, dst4, zrows, zdeg, ones)
    return run(table, src4, dst4, zrows)


BN = 1024
NGRID = NP // BN


def _tc1_body(acc0, acc1, deg0, deg1, x, w1l, b1l, w1r, w2l, h1_out, y2_out):
    acc = acc0[0] + acc1[0]
    deg = jnp.maximum(deg0[...] + deg1[...], 1.0)
    mean = acc / deg[:, None]
    h1 = lax.dot_general(mean, w1l[...], (((1,), (1,)), ((), ())),
                         preferred_element_type=jnp.float32)
    h1 = h1 + b1l[...] + lax.dot_general(x[...], w1r[...],
                                         (((1,), (1,)), ((), ())),
                                         preferred_element_type=jnp.float32)
    h1 = jnp.maximum(h1, 0.0)
    h1_out[...] = h1
    y2_out[...] = lax.dot_general(h1, w2l[...], (((1,), (1,)), ((), ())),
                                  preferred_element_type=jnp.float32)


def _tc_layer1(accp, degf, x, w1l, b1l, w1r, w2l):
    return pl.pallas_call(
        _tc1_body,
        grid=(NGRID,),
        in_specs=[
            pl.BlockSpec((1, BN, 128), lambda i: (0, i, 0)),
            pl.BlockSpec((1, BN, 128), lambda i: (1, i, 0)),
            pl.BlockSpec((BN,), lambda i: (i,)),
            pl.BlockSpec((BN,), lambda i: (NGRID + i,)),
            pl.BlockSpec((BN, 128), lambda i: (i, 0)),
            pl.BlockSpec((128, 128), lambda i: (0, 0)),
            pl.BlockSpec((1, 128), lambda i: (0, 0)),
            pl.BlockSpec((128, 128), lambda i: (0, 0)),
            pl.BlockSpec((32, 128), lambda i: (0, 0)),
        ],
        out_specs=[
            pl.BlockSpec((BN, 128), lambda i: (i, 0)),
            pl.BlockSpec((BN, 32), lambda i: (i, 0)),
        ],
        out_shape=[
            jax.ShapeDtypeStruct((N, 128), jnp.float32),
            jax.ShapeDtypeStruct((NP, 32), jnp.float32),
        ],
    )(accp, accp, degf, degf, x, w1l, b1l, w1r, w2l)


def _tc2_body(acc0, acc1, deg0, deg1, h1, w2r, b2l, wfc, bfc, out, psum):
    i = pl.program_id(0)
    acc = acc0[0] + acc1[0]
    deg = jnp.maximum(deg0[...] + deg1[...], 1.0)
    h2 = acc / deg[:, None] + b2l[...] + lax.dot_general(
        h1[...], w2r[...], (((1,), (1,)), ((), ())),
        preferred_element_type=jnp.float32)
    h2 = jnp.maximum(h2, 0.0)
    rows = i * BN + lax.broadcasted_iota(jnp.int32, (BN, 1), 0)
    h2 = jnp.where(rows < N, h2, 0.0)
    blk = jnp.sum(h2, axis=0, keepdims=True)

    @pl.when(i == 0)
    def _():
        psum[...] = blk

    @pl.when(i > 0)
    def _():
        psum[...] = psum[...] + blk

    @pl.when(i == NGRID - 1)
    def _():
        g = psum[...] / float(N)
        logits = lax.dot_general(g, wfc[...], (((1,), (1,)), ((), ())),
                                 preferred_element_type=jnp.float32) + bfc[...]
        m = jnp.max(logits)
        e = jnp.exp(logits - m)
        out[...] = e / jnp.sum(e)


def _tc_layer2(accp, degf, h1, w2r, b2l, wfc, bfc):
    return pl.pallas_call(
        _tc2_body,
        grid=(NGRID,),
        in_specs=[
            pl.BlockSpec((1, BN, 32), lambda i: (0, i, 0)),
            pl.BlockSpec((1, BN, 32), lambda i: (1, i, 0)),
            pl.BlockSpec((BN,), lambda i: (i,)),
            pl.BlockSpec((BN,), lambda i: (NGRID + i,)),
            pl.BlockSpec((BN, 128), lambda i: (i, 0)),
            pl.BlockSpec((32, 128), lambda i: (0, 0)),
            pl.BlockSpec((1, 32), lambda i: (0, 0)),
            pl.BlockSpec((16, 32), lambda i: (0, 0)),
            pl.BlockSpec((1, 16), lambda i: (0, 0)),
        ],
        out_specs=pl.BlockSpec((1, 16), lambda i: (0, 0)),
        out_shape=jax.ShapeDtypeStruct((1, 16), jnp.float32),
        scratch_shapes=[pltpu.VMEM((1, 32), jnp.float32)],
    )(accp, accp, degf, degf, h1, w2r, b2l, wfc, bfc)


def kernel(x, edge_index, w1_l, b1_l, w1_r, w2_l, b2_l, w2_r, w_fc, b_fc):
    src4 = jnp.pad(edge_index[0], (0, EPAD - E)).reshape(NW, NBLK, IB, CH)
    dst4 = jnp.pad(edge_index[1], (0, EPAD - E),
                   constant_values=N).reshape(NW, NBLK, IB, CH)
    xp = jnp.pad(x, ((0, NP - N), (0, 0)))
    z128 = jnp.zeros((RPT, 128), jnp.float32)
    z32 = jnp.zeros((RPT, 32), jnp.float32)
    z16 = jnp.zeros((RPT, 16), jnp.float32)
    ones = jnp.ones((CH, 16), jnp.float32)

    acc1p, degf = _sc_aggregate(xp, src4, dst4, z128, True, z16, ones)
    acc1p = acc1p.reshape(NC, NP, 128)

    h1, y2 = _tc_layer1(acc1p, degf, x, w1_l, b1_l.reshape(1, 128), w1_r, w2_l)

    (acc2p,) = _sc_aggregate(y2, src4, dst4, z32, False)
    acc2p = acc2p.reshape(NC, NP, 32)

    return _tc_layer2(acc2p, degf, h1, w2_r, b2_l.reshape(1, 32),
                      w_fc, b_fc.reshape(1, 16))


# speedup vs baseline: 1.0003x; 1.0003x over previous
import jax
import jax.numpy as jnp
from jax import lax
from jax.experimental import pallas as pl
from jax.experimental.pallas import tpu as pltpu
from jax.experimental.pallas import tpu_sc as plsc

N = 10000
E = 320000
NC = 2
NS = 16
NW = NC * NS
NP = 10240
RPT = NP // NS
CH = 128
EPW = NP
EPAD = NW * EPW
NCHUNK = EPW // CH
NBLK = 10
IB = NCHUNK // NBLK


def _sc_aggregate(table, src4, dst4, zrows, with_deg, zdeg=None, ones=None):
    D = table.shape[1]
    mesh = plsc.VectorSubcoreMesh(
        core_axis_name="c", subcore_axis_name="s", num_cores=NC, num_subcores=NS
    )
    out_type = [jax.ShapeDtypeStruct((NC * NP, D), jnp.float32)]
    if with_deg:
        out_type.append(jax.ShapeDtypeStruct((NC * NP,), jnp.float32))
    scratch = {
        "acc_sh": pltpu.VMEM_SHARED((NP, D), jnp.float32),
        "src_v": pltpu.VMEM((IB, CH), jnp.int32),
        "dst_v": pltpu.VMEM((IB, CH), jnp.int32),
        "dsem": pltpu.SemaphoreType.DMA,
    }
    RB = 2 if D > 64 else 4
    for r in range(RB):
        scratch[f"rows_v{r}"] = pltpu.VMEM((CH, D), jnp.float32)
        scratch[f"gsem{r}"] = pltpu.SemaphoreType.DMA
        scratch[f"ssem{r}"] = pltpu.SemaphoreType.DMA
    if with_deg:
        scratch["deg_sh"] = pltpu.VMEM_SHARED((NP, 16), jnp.float32)
        scratch["ones_v"] = pltpu.VMEM((CH, 16), jnp.float32)
        scratch["dbounce_v"] = pltpu.VMEM((64, 16), jnp.float32)
        scratch["degf_v"] = pltpu.VMEM((RPT,), jnp.float32)

    def body(*refs, acc_sh, src_v, dst_v, dsem, deg_sh=None, ones_v=None,
             dbounce_v=None, degf_v=None, **ring):
        if with_deg:
            (table_h, src_h, dst_h, zrows_h, zdeg_h, ones_h,
             acc_out, deg_out) = refs
        else:
            (table_h, src_h, dst_h, zrows_h, acc_out) = refs
        c = lax.axis_index("c")
        s = lax.axis_index("s")
        wid = c * NS + s

        pltpu.sync_copy(zrows_h, acc_sh.at[pl.ds(s * RPT, RPT)])
        if with_deg:
            pltpu.sync_copy(zdeg_h, deg_sh.at[pl.ds(s * RPT, RPT)])
            pltpu.sync_copy(ones_h, ones_v)
        plsc.subcore_barrier()

        rows = [ring[f"rows_v{r}"] for r in range(RB)]
        gsems = [ring[f"gsem{r}"] for r in range(RB)]
        ssems = [ring[f"ssem{r}"] for r in range(RB)]

        def blk_step(b, carry):
            pltpu.sync_copy(src_h.at[wid, b], src_v)
            pltpu.sync_copy(dst_h.at[wid, b], dst_v)
            gd = [None] * IB
            sd = [None] * IB
            gd[0] = pltpu.async_copy(table_h.at[src_v.at[0]], rows[0], gsems[0])
            deg_d = []
            for j in range(IB):
                if j + 1 < IB:
                    if j + 1 >= RB:
                        sd[j + 1 - RB].wait()
                    gd[j + 1] = pltpu.async_copy(
                        table_h.at[src_v.at[j + 1]], rows[(j + 1) % RB],
                        gsems[(j + 1) % RB])
                gd[j].wait()
                sd[j] = pltpu.async_copy(rows[j % RB],
                                         acc_sh.at[dst_v.at[j]],
                                         ssems[j % RB], add=True)
                if with_deg:
                    deg_d.append(pltpu.async_copy(
                        ones_v, deg_sh.at[dst_v.at[j]], dsem, add=True))
            for t in range(max(0, IB - RB), IB):
                sd[t].wait()
            for dd in deg_d:
                dd.wait()
            return carry

        lax.fori_loop(0, NBLK, blk_step, 0)
        plsc.subcore_barrier()

        pltpu.sync_copy(acc_sh.at[pl.ds(s * RPT, RPT)],
                        acc_out.at[pl.ds(c * NP + s * RPT, RPT)])
        if with_deg:
            iota16 = lax.iota(jnp.int32, 16)

            def tr_step(p, carry):
                pltpu.sync_copy(deg_sh.at[pl.ds(s * RPT + p * 64, 64)],
                                dbounce_v)
                for t in range(4):
                    vec = jnp.zeros((16,), jnp.float32)
                    for k in range(16):
                        vec = jnp.where(iota16 == k,
                                        dbounce_v[16 * t + k, :], vec)
                    degf_v[pl.ds(p * 64 + t * 16, 16)] = vec
                return carry

            lax.fori_loop(0, RPT // 64, tr_step, 0)
            pltpu.sync_copy(degf_v, deg_out.at[pl.ds(c * NP + s * RPT, RPT)])

    run = pl.kernel(body, out_type=out_type, mesh=mesh, scratch_types=scratch,
                    compiler_params=pltpu.CompilerParams(
                        use_tc_tiling_on_sc=False))
    if with_deg:
        return run(table, src4, dst4, zrows, zdeg, ones)
    return run(table, src4, dst4, zrows)


BN = 1024
NGRID = NP // BN


def _tc1_body(acc0, acc1, deg0, deg1, x, w1l, b1l, w1r, w2l, h1_out, y2_out):
    acc = acc0[0] + acc1[0]
    deg = jnp.maximum(deg0[...] + deg1[...], 1.0)
    mean = acc / deg[:, None]
    h1 = lax.dot_general(mean, w1l[...], (((1,), (1,)), ((), ())),
                         preferred_element_type=jnp.float32)
    h1 = h1 + b1l[...] + lax.dot_general(x[...], w1r[...],
                                         (((1,), (1,)), ((), ())),
                                         preferred_element_type=jnp.float32)
    h1 = jnp.maximum(h1, 0.0)
    h1_out[...] = h1
    y2_out[...] = lax.dot_general(h1, w2l[...], (((1,), (1,)), ((), ())),
                                  preferred_element_type=jnp.float32)


def _tc_layer1(accp, degf, x, w1l, b1l, w1r, w2l):
    return pl.pallas_call(
        _tc1_body,
        grid=(NGRID,),
        in_specs=[
            pl.BlockSpec((1, BN, 128), lambda i: (0, i, 0)),
            pl.BlockSpec((1, BN, 128), lambda i: (1, i, 0)),
            pl.BlockSpec((BN,), lambda i: (i,)),
            pl.BlockSpec((BN,), lambda i: (NGRID + i,)),
            pl.BlockSpec((BN, 128), lambda i: (i, 0)),
            pl.BlockSpec((128, 128), lambda i: (0, 0)),
            pl.BlockSpec((1, 128), lambda i: (0, 0)),
            pl.BlockSpec((128, 128), lambda i: (0, 0)),
            pl.BlockSpec((32, 128), lambda i: (0, 0)),
        ],
        out_specs=[
            pl.BlockSpec((BN, 128), lambda i: (i, 0)),
            pl.BlockSpec((BN, 32), lambda i: (i, 0)),
        ],
        out_shape=[
            jax.ShapeDtypeStruct((N, 128), jnp.float32),
            jax.ShapeDtypeStruct((NP, 32), jnp.float32),
        ],
    )(accp, accp, degf, degf, x, w1l, b1l, w1r, w2l)


def _tc2_body(acc0, acc1, deg0, deg1, h1, w2r, b2l, wfc, bfc, out, psum):
    i = pl.program_id(0)
    acc = acc0[0] + acc1[0]
    deg = jnp.maximum(deg0[...] + deg1[...], 1.0)
    h2 = acc / deg[:, None] + b2l[...] + lax.dot_general(
        h1[...], w2r[...], (((1,), (1,)), ((), ())),
        preferred_element_type=jnp.float32)
    h2 = jnp.maximum(h2, 0.0)
    rows = i * BN + lax.broadcasted_iota(jnp.int32, (BN, 1), 0)
    h2 = jnp.where(rows < N, h2, 0.0)
    blk = jnp.sum(h2, axis=0, keepdims=True)

    @pl.when(i == 0)
    def _():
        psum[...] = blk

    @pl.when(i > 0)
    def _():
        psum[...] = psum[...] + blk

    @pl.when(i == NGRID - 1)
    def _():
        g = psum[...] / float(N)
        logits = lax.dot_general(g, wfc[...], (((1,), (1,)), ((), ())),
                                 preferred_element_type=jnp.float32) + bfc[...]
        m = jnp.max(logits)
        e = jnp.exp(logits - m)
        out[...] = e / jnp.sum(e)


def _tc_layer2(accp, degf, h1, w2r, b2l, wfc, bfc):
    return pl.pallas_call(
        _tc2_body,
        grid=(NGRID,),
        in_specs=[
            pl.BlockSpec((1, BN, 32), lambda i: (0, i, 0)),
            pl.BlockSpec((1, BN, 32), lambda i: (1, i, 0)),
            pl.BlockSpec((BN,), lambda i: (i,)),
            pl.BlockSpec((BN,), lambda i: (NGRID + i,)),
            pl.BlockSpec((BN, 128), lambda i: (i, 0)),
            pl.BlockSpec((32, 128), lambda i: (0, 0)),
            pl.BlockSpec((1, 32), lambda i: (0, 0)),
            pl.BlockSpec((16, 32), lambda i: (0, 0)),
            pl.BlockSpec((1, 16), lambda i: (0, 0)),
        ],
        out_specs=pl.BlockSpec((1, 16), lambda i: (0, 0)),
        out_shape=jax.ShapeDtypeStruct((1, 16), jnp.float32),
        scratch_shapes=[pltpu.VMEM((1, 32), jnp.float32)],
    )(accp, accp, degf, degf, h1, w2r, b2l, wfc, bfc)


def kernel(x, edge_index, w1_l, b1_l, w1_r, w2_l, b2_l, w2_r, w_fc, b_fc):
    src4 = jnp.pad(edge_index[0], (0, EPAD - E)).reshape(NW, NBLK, IB, CH)
    pad_dst = N + jnp.arange(EPAD - E, dtype=jnp.int32) % (NP - N)
    dst4 = jnp.concatenate([edge_index[1], pad_dst]).reshape(NW, NBLK, IB, CH)
    xp = jnp.pad(x, ((0, NP - N), (0, 0)))
    z128 = jnp.zeros((RPT, 128), jnp.float32)
    z32 = jnp.zeros((RPT, 32), jnp.float32)
    z16 = jnp.zeros((RPT, 16), jnp.float32)
    ones = jnp.ones((CH, 16), jnp.float32)

    acc1p, degf = _sc_aggregate(xp, src4, dst4, z128, True, z16, ones)
    acc1p = acc1p.reshape(NC, NP, 128)

    h1, y2 = _tc_layer1(acc1p, degf, x, w1_l, b1_l.reshape(1, 128), w1_r, w2_l)

    (acc2p,) = _sc_aggregate(y2, src4, dst4, z32, False)
    acc2p = acc2p.reshape(NC, NP, 32)

    return _tc_layer2(acc2p, degf, h1, w2_r, b2_l.reshape(1, 32),
                      w_fc, b_fc.reshape(1, 16))


# speedup vs baseline: 2.6855x; 2.6847x over previous
import jax
import jax.numpy as jnp
from jax import lax
from jax.experimental import pallas as pl
from jax.experimental.pallas import tpu as pltpu
from jax.experimental.pallas import tpu_sc as plsc

N = 10000
E = 320000
NC = 2
NS = 16
NW = NC * NS
NP = 10240
RPT = NP // NS
CH = 128
EPW = NP
EPAD = NW * EPW
NCHUNK = EPW // CH
NBLK = 10
IB = NCHUNK // NBLK


def _sc_aggregate(table, src4, dst4, zrows, with_deg, zdeg=None, ones=None):
    D = table.shape[1]
    mesh = plsc.VectorSubcoreMesh(
        core_axis_name="c", subcore_axis_name="s", num_cores=NC, num_subcores=NS
    )
    out_type = [jax.ShapeDtypeStruct((NC * NP, D), jnp.float32)]
    if with_deg:
        out_type.append(jax.ShapeDtypeStruct((NC * NP,), jnp.float32))
    scratch = {
        "acc_sh": pltpu.VMEM_SHARED((NP, D), jnp.float32),
        "src_v": pltpu.VMEM((IB, CH), jnp.int32),
        "dst_v": pltpu.VMEM((IB, CH), jnp.int32),
        "dsem": pltpu.SemaphoreType.DMA,
    }
    RB = 2 if D > 64 else 4
    for r in range(RB):
        scratch[f"rows_v{r}"] = pltpu.VMEM((CH, D), jnp.float32)
        scratch[f"gsem{r}"] = pltpu.SemaphoreType.DMA
        scratch[f"ssem{r}"] = pltpu.SemaphoreType.DMA
    if with_deg:
        scratch["deg_sh"] = pltpu.VMEM_SHARED((NP, 16), jnp.float32)
        scratch["ones_v"] = pltpu.VMEM((CH, 16), jnp.float32)
        scratch["dbounce_v"] = pltpu.VMEM((64, 16), jnp.float32)
        scratch["degf_v"] = pltpu.VMEM((RPT,), jnp.float32)

    def body(*refs, acc_sh, src_v, dst_v, dsem, deg_sh=None, ones_v=None,
             dbounce_v=None, degf_v=None, **ring):
        if with_deg:
            (table_h, src_h, dst_h, zrows_h, zdeg_h, ones_h,
             acc_out, deg_out) = refs
        else:
            (table_h, src_h, dst_h, zrows_h, acc_out) = refs
        c = lax.axis_index("c")
        s = lax.axis_index("s")
        wid = c * NS + s

        pltpu.sync_copy(zrows_h, acc_sh.at[pl.ds(s * RPT, RPT)])
        if with_deg:
            pltpu.sync_copy(zdeg_h, deg_sh.at[pl.ds(s * RPT, RPT)])
            pltpu.sync_copy(ones_h, ones_v)
        plsc.subcore_barrier()

        rows = [ring[f"rows_v{r}"] for r in range(RB)]
        gsems = [ring[f"gsem{r}"] for r in range(RB)]
        ssems = [ring[f"ssem{r}"] for r in range(RB)]

        def blk_step(b, carry):
            pltpu.sync_copy(src_h.at[wid, b], src_v)
            pltpu.sync_copy(dst_h.at[wid, b], dst_v)
            gd = [None] * IB
            sd = [None] * IB
            gd[0] = pltpu.async_copy(table_h.at[src_v.at[0]], rows[0], gsems[0])
            deg_d = []
            for j in range(IB):
                if j + 1 < IB:
                    if j + 1 >= RB:
                        sd[j + 1 - RB].wait()
                    gd[j + 1] = pltpu.async_copy(
                        table_h.at[src_v.at[j + 1]], rows[(j + 1) % RB],
                        gsems[(j + 1) % RB])
                gd[j].wait()
                sd[j] = pltpu.async_copy(rows[j % RB],
                                         acc_sh.at[dst_v.at[j]],
                                         ssems[j % RB], add=True)
                if with_deg:
                    deg_d.append(pltpu.async_copy(
                        ones_v, deg_sh.at[dst_v.at[j]], dsem, add=True))
            for t in range(max(0, IB - RB), IB):
                sd[t].wait()
            for dd in deg_d:
                dd.wait()
            return carry

        lax.fori_loop(0, NBLK, blk_step, 0)
        plsc.subcore_barrier()

        pltpu.sync_copy(acc_sh.at[pl.ds(s * RPT, RPT)],
                        acc_out.at[pl.ds(c * NP + s * RPT, RPT)])
        if with_deg:
            iota16 = lax.iota(jnp.int32, 16)

            def tr_step(p, carry):
                pltpu.sync_copy(deg_sh.at[pl.ds(s * RPT + p * 64, 64)],
                                dbounce_v)
                for t in range(4):
                    vec = jnp.zeros((16,), jnp.float32)
                    for k in range(16):
                        vec = jnp.where(iota16 == k,
                                        dbounce_v[16 * t + k, :], vec)
                    degf_v[pl.ds(p * 64 + t * 16, 16)] = vec
                return carry

            lax.fori_loop(0, RPT // 64, tr_step, 0)
            pltpu.sync_copy(degf_v, deg_out.at[pl.ds(c * NP + s * RPT, RPT)])

    run = pl.kernel(body, out_type=out_type, mesh=mesh, scratch_types=scratch,
                    compiler_params=pltpu.CompilerParams(
                        use_tc_tiling_on_sc=False))
    if with_deg:
        return run(table, src4, dst4, zrows, zdeg, ones)
    return run(table, src4, dst4, zrows)


BN = 1024
NGRID = NP // BN


def _tc1_body(acc0, acc1, deg0, deg1, x, w1l, b1l, w1r, w2l, h1_out, y2_out):
    acc = acc0[0] + acc1[0]
    deg = jnp.maximum(deg0[...] + deg1[...], 1.0)
    mean = acc / deg[:, None]
    h1 = lax.dot_general(mean, w1l[...], (((1,), (1,)), ((), ())),
                         preferred_element_type=jnp.float32)
    h1 = h1 + b1l[...] + lax.dot_general(x[...], w1r[...],
                                         (((1,), (1,)), ((), ())),
                                         preferred_element_type=jnp.float32)
    h1 = jnp.maximum(h1, 0.0)
    h1_out[...] = h1
    y2_out[...] = lax.dot_general(h1, w2l[...], (((1,), (1,)), ((), ())),
                                  preferred_element_type=jnp.float32)


def _tc_layer1(accp, degf, x, w1l, b1l, w1r, w2l):
    return pl.pallas_call(
        _tc1_body,
        grid=(NGRID,),
        in_specs=[
            pl.BlockSpec((1, BN, 128), lambda i: (0, i, 0)),
            pl.BlockSpec((1, BN, 128), lambda i: (1, i, 0)),
            pl.BlockSpec((BN,), lambda i: (i,)),
            pl.BlockSpec((BN,), lambda i: (NGRID + i,)),
            pl.BlockSpec((BN, 128), lambda i: (i, 0)),
            pl.BlockSpec((128, 128), lambda i: (0, 0)),
            pl.BlockSpec((1, 128), lambda i: (0, 0)),
            pl.BlockSpec((128, 128), lambda i: (0, 0)),
            pl.BlockSpec((32, 128), lambda i: (0, 0)),
        ],
        out_specs=[
            pl.BlockSpec((BN, 128), lambda i: (i, 0)),
            pl.BlockSpec((BN, 32), lambda i: (i, 0)),
        ],
        out_shape=[
            jax.ShapeDtypeStruct((N, 128), jnp.float32),
            jax.ShapeDtypeStruct((NP, 32), jnp.float32),
        ],
    )(accp, accp, degf, degf, x, w1l, b1l, w1r, w2l)


def _tc2_body(acc0, acc1, deg0, deg1, h1, w2r, b2l, wfc, bfc, out, psum):
    i = pl.program_id(0)
    acc = acc0[0] + acc1[0]
    deg = jnp.maximum(deg0[...] + deg1[...], 1.0)
    h2 = acc / deg[:, None] + b2l[...] + lax.dot_general(
        h1[...], w2r[...], (((1,), (1,)), ((), ())),
        preferred_element_type=jnp.float32)
    h2 = jnp.maximum(h2, 0.0)
    rows = i * BN + lax.broadcasted_iota(jnp.int32, (BN, 1), 0)
    h2 = jnp.where(rows < N, h2, 0.0)
    blk = jnp.sum(h2, axis=0, keepdims=True)

    @pl.when(i == 0)
    def _():
        psum[...] = blk

    @pl.when(i > 0)
    def _():
        psum[...] = psum[...] + blk

    @pl.when(i == NGRID - 1)
    def _():
        g = psum[...] / float(N)
        logits = lax.dot_general(g, wfc[...], (((1,), (1,)), ((), ())),
                                 preferred_element_type=jnp.float32) + bfc[...]
        m = jnp.max(logits)
        e = jnp.exp(logits - m)
        out[...] = e / jnp.sum(e)


def _tc_layer2(accp, degf, h1, w2r, b2l, wfc, bfc):
    return pl.pallas_call(
        _tc2_body,
        grid=(NGRID,),
        in_specs=[
            pl.BlockSpec((1, BN, 32), lambda i: (0, i, 0)),
            pl.BlockSpec((1, BN, 32), lambda i: (1, i, 0)),
            pl.BlockSpec((BN,), lambda i: (i,)),
            pl.BlockSpec((BN,), lambda i: (NGRID + i,)),
            pl.BlockSpec((BN, 128), lambda i: (i, 0)),
            pl.BlockSpec((32, 128), lambda i: (0, 0)),
            pl.BlockSpec((1, 32), lambda i: (0, 0)),
            pl.BlockSpec((16, 32), lambda i: (0, 0)),
            pl.BlockSpec((1, 16), lambda i: (0, 0)),
        ],
        out_specs=pl.BlockSpec((1, 16), lambda i: (0, 0)),
        out_shape=jax.ShapeDtypeStruct((1, 16), jnp.float32),
        scratch_shapes=[pltpu.VMEM((1, 32), jnp.float32)],
    )(accp, accp, degf, degf, h1, w2r, b2l, wfc, bfc)


def kernel(x, edge_index, w1_l, b1_l, w1_r, w2_l, b2_l, w2_r, w_fc, b_fc):
    pad_iota = jnp.arange(EPAD - E, dtype=jnp.int32)
    src4 = jnp.concatenate([edge_index[0],
                            pad_iota % N]).reshape(NW, NBLK, IB, CH)
    dst4 = jnp.concatenate([edge_index[1],
                            N + pad_iota % (NP - N)]).reshape(NW, NBLK, IB, CH)
    xp = jnp.pad(x, ((0, NP - N), (0, 0)))
    z128 = jnp.zeros((RPT, 128), jnp.float32)
    z32 = jnp.zeros((RPT, 32), jnp.float32)
    z16 = jnp.zeros((RPT, 16), jnp.float32)
    ones = jnp.ones((CH, 16), jnp.float32)

    acc1p, degf = _sc_aggregate(xp, src4, dst4, z128, True, z16, ones)
    acc1p = acc1p.reshape(NC, NP, 128)

    h1, y2 = _tc_layer1(acc1p, degf, x, w1_l, b1_l.reshape(1, 128), w1_r, w2_l)

    (acc2p,) = _sc_aggregate(y2, src4, dst4, z32, False)
    acc2p = acc2p.reshape(NC, NP, 32)

    return _tc_layer2(acc2p, degf, h1, w2_r, b2_l.reshape(1, 32),
                      w_fc, b_fc.reshape(1, 16))


# speedup vs baseline: 2.7539x; 1.0255x over previous
import jax
import jax.numpy as jnp
from jax import lax
from jax.experimental import pallas as pl
from jax.experimental.pallas import tpu as pltpu
from jax.experimental.pallas import tpu_sc as plsc

N = 10000
E = 320000
NC = 2
NS = 16
NW = NC * NS
NP = 10240
RPT = NP // NS
CH = 128
EPW = NP
EPAD = NW * EPW
NCHUNK = EPW // CH
NBLK1 = 10
IB1 = NCHUNK // NBLK1
NBLK2 = 5
IB2 = NCHUNK // NBLK2


def _sc_aggregate(table, src4, dst4, zrows, with_deg, zdeg=None, ones=None,
                  nblk=NBLK1, ib=IB1):
    D = table.shape[1]
    mesh = plsc.VectorSubcoreMesh(
        core_axis_name="c", subcore_axis_name="s", num_cores=NC, num_subcores=NS
    )
    out_type = [jax.ShapeDtypeStruct((NC * NP, D), jnp.float32)]
    if with_deg:
        out_type.append(jax.ShapeDtypeStruct((NC * NP,), jnp.float32))
    scratch = {
        "acc_sh": pltpu.VMEM_SHARED((NP, D), jnp.float32),
        "src_v": pltpu.VMEM((ib, CH), jnp.int32),
        "dst_v": pltpu.VMEM((ib, CH), jnp.int32),
        "dsem": pltpu.SemaphoreType.DMA,
    }
    RB = 2 if D > 64 else 4
    for r in range(RB):
        scratch[f"rows_v{r}"] = pltpu.VMEM((CH, D), jnp.float32)
        scratch[f"gsem{r}"] = pltpu.SemaphoreType.DMA
        scratch[f"ssem{r}"] = pltpu.SemaphoreType.DMA
    if with_deg:
        scratch["deg_sh"] = pltpu.VMEM_SHARED((NP, 16), jnp.float32)
        scratch["ones_v"] = pltpu.VMEM((CH, 16), jnp.float32)
        scratch["dbounce_v"] = pltpu.VMEM((64, 16), jnp.float32)
        scratch["degf_v"] = pltpu.VMEM((RPT,), jnp.float32)

    def body(*refs, acc_sh, src_v, dst_v, dsem, deg_sh=None, ones_v=None,
             dbounce_v=None, degf_v=None, **ring):
        if with_deg:
            (table_h, src_h, dst_h, zrows_h, zdeg_h, ones_h,
             acc_out, deg_out) = refs
        else:
            (table_h, src_h, dst_h, zrows_h, acc_out) = refs
        c = lax.axis_index("c")
        s = lax.axis_index("s")
        wid = c * NS + s

        pltpu.sync_copy(zrows_h, acc_sh.at[pl.ds(s * RPT, RPT)])
        if with_deg:
            pltpu.sync_copy(zdeg_h, deg_sh.at[pl.ds(s * RPT, RPT)])
            pltpu.sync_copy(ones_h, ones_v)
        plsc.subcore_barrier()

        rows = [ring[f"rows_v{r}"] for r in range(RB)]
        gsems = [ring[f"gsem{r}"] for r in range(RB)]
        ssems = [ring[f"ssem{r}"] for r in range(RB)]

        def blk_step(b, carry):
            pltpu.sync_copy(src_h.at[wid, b], src_v)
            pltpu.sync_copy(dst_h.at[wid, b], dst_v)
            gd = [None] * ib
            sd = [None] * ib
            gd[0] = pltpu.async_copy(table_h.at[src_v.at[0]], rows[0], gsems[0])
            deg_d = []
            for j in range(ib):
                if j + 1 < ib:
                    if j + 1 >= RB:
                        sd[j + 1 - RB].wait()
                    gd[j + 1] = pltpu.async_copy(
                        table_h.at[src_v.at[j + 1]], rows[(j + 1) % RB],
                        gsems[(j + 1) % RB])
                gd[j].wait()
                sd[j] = pltpu.async_copy(rows[j % RB],
                                         acc_sh.at[dst_v.at[j]],
                                         ssems[j % RB], add=True)
                if with_deg:
                    deg_d.append(pltpu.async_copy(
                        ones_v, deg_sh.at[dst_v.at[j]], dsem, add=True))
            for t in range(max(0, ib - RB), ib):
                sd[t].wait()
            for dd in deg_d:
                dd.wait()
            return carry

        lax.fori_loop(0, nblk, blk_step, 0)
        plsc.subcore_barrier()

        pltpu.sync_copy(acc_sh.at[pl.ds(s * RPT, RPT)],
                        acc_out.at[pl.ds(c * NP + s * RPT, RPT)])
        if with_deg:
            iota16 = lax.iota(jnp.int32, 16)

            def tr_step(p, carry):
                pltpu.sync_copy(deg_sh.at[pl.ds(s * RPT + p * 64, 64)],
                                dbounce_v)
                for t in range(4):
                    vec = jnp.zeros((16,), jnp.float32)
                    for k in range(16):
                        vec = jnp.where(iota16 == k,
                                        dbounce_v[16 * t + k, :], vec)
                    degf_v[pl.ds(p * 64 + t * 16, 16)] = vec
                return carry

            lax.fori_loop(0, RPT // 64, tr_step, 0)
            pltpu.sync_copy(degf_v, deg_out.at[pl.ds(c * NP + s * RPT, RPT)])

    run = pl.kernel(body, out_type=out_type, mesh=mesh, scratch_types=scratch,
                    compiler_params=pltpu.CompilerParams(
                        use_tc_tiling_on_sc=False))
    if with_deg:
        return run(table, src4, dst4, zrows, zdeg, ones)
    return run(table, src4, dst4, zrows)


BN = 1024
NGRID = NP // BN


def _tc1_body(acc0, acc1, deg0, deg1, x, w1l, b1l, w1r, w2l, h1_out, y2_out):
    acc = acc0[0] + acc1[0]
    deg = jnp.maximum(deg0[...] + deg1[...], 1.0)
    mean = acc / deg[:, None]
    h1 = lax.dot_general(mean, w1l[...], (((1,), (1,)), ((), ())),
                         preferred_element_type=jnp.float32)
    h1 = h1 + b1l[...] + lax.dot_general(x[...], w1r[...],
                                         (((1,), (1,)), ((), ())),
                                         preferred_element_type=jnp.float32)
    h1 = jnp.maximum(h1, 0.0)
    h1_out[...] = h1
    y2_out[...] = lax.dot_general(h1, w2l[...], (((1,), (1,)), ((), ())),
                                  preferred_element_type=jnp.float32)


def _tc_layer1(accp, degf, x, w1l, b1l, w1r, w2l):
    return pl.pallas_call(
        _tc1_body,
        grid=(NGRID,),
        in_specs=[
            pl.BlockSpec((1, BN, 128), lambda i: (0, i, 0)),
            pl.BlockSpec((1, BN, 128), lambda i: (1, i, 0)),
            pl.BlockSpec((BN,), lambda i: (i,)),
            pl.BlockSpec((BN,), lambda i: (NGRID + i,)),
            pl.BlockSpec((BN, 128), lambda i: (i, 0)),
            pl.BlockSpec((128, 128), lambda i: (0, 0)),
            pl.BlockSpec((1, 128), lambda i: (0, 0)),
            pl.BlockSpec((128, 128), lambda i: (0, 0)),
            pl.BlockSpec((32, 128), lambda i: (0, 0)),
        ],
        out_specs=[
            pl.BlockSpec((BN, 128), lambda i: (i, 0)),
            pl.BlockSpec((BN, 32), lambda i: (i, 0)),
        ],
        out_shape=[
            jax.ShapeDtypeStruct((N, 128), jnp.float32),
            jax.ShapeDtypeStruct((NP, 32), jnp.float32),
        ],
    )(accp, accp, degf, degf, x, w1l, b1l, w1r, w2l)


def _tc2_body(acc0, acc1, deg0, deg1, h1, w2r, b2l, wfc, bfc, out, psum):
    i = pl.program_id(0)
    acc = acc0[0] + acc1[0]
    deg = jnp.maximum(deg0[...] + deg1[...], 1.0)
    h2 = acc / deg[:, None] + b2l[...] + lax.dot_general(
        h1[...], w2r[...], (((1,), (1,)), ((), ())),
        preferred_element_type=jnp.float32)
    h2 = jnp.maximum(h2, 0.0)
    rows = i * BN + lax.broadcasted_iota(jnp.int32, (BN, 1), 0)
    h2 = jnp.where(rows < N, h2, 0.0)
    blk = jnp.sum(h2, axis=0, keepdims=True)

    @pl.when(i == 0)
    def _():
        psum[...] = blk

    @pl.when(i > 0)
    def _():
        psum[...] = psum[...] + blk

    @pl.when(i == NGRID - 1)
    def _():
        g = psum[...] / float(N)
        logits = lax.dot_general(g, wfc[...], (((1,), (1,)), ((), ())),
                                 preferred_element_type=jnp.float32) + bfc[...]
        m = jnp.max(logits)
        e = jnp.exp(logits - m)
        out[...] = e / jnp.sum(e)


def _tc_layer2(accp, degf, h1, w2r, b2l, wfc, bfc):
    return pl.pallas_call(
        _tc2_body,
        grid=(NGRID,),
        in_specs=[
            pl.BlockSpec((1, BN, 32), lambda i: (0, i, 0)),
            pl.BlockSpec((1, BN, 32), lambda i: (1, i, 0)),
            pl.BlockSpec((BN,), lambda i: (i,)),
            pl.BlockSpec((BN,), lambda i: (NGRID + i,)),
            pl.BlockSpec((BN, 128), lambda i: (i, 0)),
            pl.BlockSpec((32, 128), lambda i: (0, 0)),
            pl.BlockSpec((1, 32), lambda i: (0, 0)),
            pl.BlockSpec((16, 32), lambda i: (0, 0)),
            pl.BlockSpec((1, 16), lambda i: (0, 0)),
        ],
        out_specs=pl.BlockSpec((1, 16), lambda i: (0, 0)),
        out_shape=jax.ShapeDtypeStruct((1, 16), jnp.float32),
        scratch_shapes=[pltpu.VMEM((1, 32), jnp.float32)],
    )(accp, accp, degf, degf, h1, w2r, b2l, wfc, bfc)


def kernel(x, edge_index, w1_l, b1_l, w1_r, w2_l, b2_l, w2_r, w_fc, b_fc):
    pad_iota = jnp.arange(EPAD - E, dtype=jnp.int32)
    srcp = jnp.concatenate([edge_index[0], pad_iota % N])
    dstp = jnp.concatenate([edge_index[1], N + pad_iota % (NP - N)])
    src1 = srcp.reshape(NW, NBLK1, IB1, CH)
    dst1 = dstp.reshape(NW, NBLK1, IB1, CH)
    src2 = srcp.reshape(NW, NBLK2, IB2, CH)
    dst2 = dstp.reshape(NW, NBLK2, IB2, CH)
    xp = jnp.pad(x, ((0, NP - N), (0, 0)))
    z128 = jnp.zeros((RPT, 128), jnp.float32)
    z32 = jnp.zeros((RPT, 32), jnp.float32)
    z16 = jnp.zeros((RPT, 16), jnp.float32)
    ones = jnp.ones((CH, 16), jnp.float32)

    acc1p, degf = _sc_aggregate(xp, src1, dst1, z128, True, z16, ones,
                                nblk=NBLK1, ib=IB1)
    acc1p = acc1p.reshape(NC, NP, 128)

    h1, y2 = _tc_layer1(acc1p, degf, x, w1_l, b1_l.reshape(1, 128), w1_r, w2_l)

    (acc2p,) = _sc_aggregate(y2, src2, dst2, z32, False, nblk=NBLK2, ib=IB2)
    acc2p = acc2p.reshape(NC, NP, 32)

    return _tc_layer2(acc2p, degf, h1, w2_r, b2_l.reshape(1, 32),
                      w_fc, b_fc.reshape(1, 16))


# speedup vs baseline: 2.7890x; 1.0128x over previous
import jax
import jax.numpy as jnp
from jax import lax
from jax.experimental import pallas as pl
from jax.experimental.pallas import tpu as pltpu
from jax.experimental.pallas import tpu_sc as plsc

N = 10000
E = 320000
NC = 2
NS = 16
NW = NC * NS
NP = 10240
RPT = NP // NS
CH = 128
EPW = NP
EPAD = NW * EPW
NCHUNK = EPW // CH
NBLK1 = 8
IB1 = NCHUNK // NBLK1
NBLK2 = 5
IB2 = NCHUNK // NBLK2


def _sc_aggregate(table, src4, dst4, zrows, with_deg, zdeg=None, ones=None,
                  nblk=NBLK1, ib=IB1):
    D = table.shape[1]
    mesh = plsc.VectorSubcoreMesh(
        core_axis_name="c", subcore_axis_name="s", num_cores=NC, num_subcores=NS
    )
    out_type = [jax.ShapeDtypeStruct((NC * NP, D), jnp.float32)]
    if with_deg:
        out_type.append(jax.ShapeDtypeStruct((NC * NP,), jnp.float32))
    scratch = {
        "acc_sh": pltpu.VMEM_SHARED((NP, D), jnp.float32),
        "src_v": pltpu.VMEM((ib, CH), jnp.int32),
        "dst_v": pltpu.VMEM((ib, CH), jnp.int32),
        "dsem": pltpu.SemaphoreType.DMA,
    }
    RB = 2 if D > 64 else 4
    for r in range(RB):
        scratch[f"rows_v{r}"] = pltpu.VMEM((CH, D), jnp.float32)
        scratch[f"gsem{r}"] = pltpu.SemaphoreType.DMA
        scratch[f"ssem{r}"] = pltpu.SemaphoreType.DMA
    if with_deg:
        scratch["deg_sh"] = pltpu.VMEM_SHARED((NP, 16), jnp.float32)
        scratch["ones_v"] = pltpu.VMEM((CH, 16), jnp.float32)
        scratch["dbounce_v"] = pltpu.VMEM((32, 16), jnp.float32)
        scratch["degf_v"] = pltpu.VMEM((RPT,), jnp.float32)

    def body(*refs, acc_sh, src_v, dst_v, dsem, deg_sh=None, ones_v=None,
             dbounce_v=None, degf_v=None, **ring):
        if with_deg:
            (table_h, src_h, dst_h, zrows_h, zdeg_h, ones_h,
             acc_out, deg_out) = refs
        else:
            (table_h, src_h, dst_h, zrows_h, acc_out) = refs
        c = lax.axis_index("c")
        s = lax.axis_index("s")
        wid = c * NS + s

        pltpu.sync_copy(zrows_h, acc_sh.at[pl.ds(s * RPT, RPT)])
        if with_deg:
            pltpu.sync_copy(zdeg_h, deg_sh.at[pl.ds(s * RPT, RPT)])
            pltpu.sync_copy(ones_h, ones_v)
        plsc.subcore_barrier()

        rows = [ring[f"rows_v{r}"] for r in range(RB)]
        gsems = [ring[f"gsem{r}"] for r in range(RB)]
        ssems = [ring[f"ssem{r}"] for r in range(RB)]

        def blk_step(b, carry):
            pltpu.sync_copy(src_h.at[wid, b], src_v)
            pltpu.sync_copy(dst_h.at[wid, b], dst_v)
            gd = [None] * ib
            sd = [None] * ib
            gd[0] = pltpu.async_copy(table_h.at[src_v.at[0]], rows[0], gsems[0])
            deg_d = []
            for j in range(ib):
                if j + 1 < ib:
                    if j + 1 >= RB:
                        sd[j + 1 - RB].wait()
                    gd[j + 1] = pltpu.async_copy(
                        table_h.at[src_v.at[j + 1]], rows[(j + 1) % RB],
                        gsems[(j + 1) % RB])
                gd[j].wait()
                sd[j] = pltpu.async_copy(rows[j % RB],
                                         acc_sh.at[dst_v.at[j]],
                                         ssems[j % RB], add=True)
                if with_deg:
                    deg_d.append(pltpu.async_copy(
                        ones_v, deg_sh.at[dst_v.at[j]], dsem, add=True))
            for t in range(max(0, ib - RB), ib):
                sd[t].wait()
            for dd in deg_d:
                dd.wait()
            return carry

        lax.fori_loop(0, nblk, blk_step, 0)
        plsc.subcore_barrier()

        pltpu.sync_copy(acc_sh.at[pl.ds(s * RPT, RPT)],
                        acc_out.at[pl.ds(c * NP + s * RPT, RPT)])
        if with_deg:
            iota16 = lax.iota(jnp.int32, 16)

            def tr_step(p, carry):
                pltpu.sync_copy(deg_sh.at[pl.ds(s * RPT + p * 32, 32)],
                                dbounce_v)
                for t in range(2):
                    vec = jnp.zeros((16,), jnp.float32)
                    for k in range(16):
                        vec = jnp.where(iota16 == k,
                                        dbounce_v[16 * t + k, :], vec)
                    degf_v[pl.ds(p * 32 + t * 16, 16)] = vec
                return carry

            lax.fori_loop(0, RPT // 32, tr_step, 0)
            pltpu.sync_copy(degf_v, deg_out.at[pl.ds(c * NP + s * RPT, RPT)])

    run = pl.kernel(body, out_type=out_type, mesh=mesh, scratch_types=scratch,
                    compiler_params=pltpu.CompilerParams(
                        use_tc_tiling_on_sc=False))
    if with_deg:
        return run(table, src4, dst4, zrows, zdeg, ones)
    return run(table, src4, dst4, zrows)


BN = 1024
NGRID = NP // BN


def _tc1_body(acc0, acc1, deg0, deg1, x, w1l, b1l, w1r, w2l, h1_out, y2_out):
    acc = acc0[0] + acc1[0]
    deg = jnp.maximum(deg0[...] + deg1[...], 1.0)
    mean = acc / deg[:, None]
    h1 = lax.dot_general(mean, w1l[...], (((1,), (1,)), ((), ())),
                         preferred_element_type=jnp.float32)
    h1 = h1 + b1l[...] + lax.dot_general(x[...], w1r[...],
                                         (((1,), (1,)), ((), ())),
                                         preferred_element_type=jnp.float32)
    h1 = jnp.maximum(h1, 0.0)
    h1_out[...] = h1
    y2_out[...] = lax.dot_general(h1, w2l[...], (((1,), (1,)), ((), ())),
                                  preferred_element_type=jnp.float32)


def _tc_layer1(accp, degf, x, w1l, b1l, w1r, w2l):
    return pl.pallas_call(
        _tc1_body,
        grid=(NGRID,),
        in_specs=[
            pl.BlockSpec((1, BN, 128), lambda i: (0, i, 0)),
            pl.BlockSpec((1, BN, 128), lambda i: (1, i, 0)),
            pl.BlockSpec((BN,), lambda i: (i,)),
            pl.BlockSpec((BN,), lambda i: (NGRID + i,)),
            pl.BlockSpec((BN, 128), lambda i: (i, 0)),
            pl.BlockSpec((128, 128), lambda i: (0, 0)),
            pl.BlockSpec((1, 128), lambda i: (0, 0)),
            pl.BlockSpec((128, 128), lambda i: (0, 0)),
            pl.BlockSpec((32, 128), lambda i: (0, 0)),
        ],
        out_specs=[
            pl.BlockSpec((BN, 128), lambda i: (i, 0)),
            pl.BlockSpec((BN, 32), lambda i: (i, 0)),
        ],
        out_shape=[
            jax.ShapeDtypeStruct((N, 128), jnp.float32),
            jax.ShapeDtypeStruct((NP, 32), jnp.float32),
        ],
    )(accp, accp, degf, degf, x, w1l, b1l, w1r, w2l)


def _tc2_body(acc0, acc1, deg0, deg1, h1, w2r, b2l, wfc, bfc, out, psum):
    i = pl.program_id(0)
    acc = acc0[0] + acc1[0]
    deg = jnp.maximum(deg0[...] + deg1[...], 1.0)
    h2 = acc / deg[:, None] + b2l[...] + lax.dot_general(
        h1[...], w2r[...], (((1,), (1,)), ((), ())),
        preferred_element_type=jnp.float32)
    h2 = jnp.maximum(h2, 0.0)
    rows = i * BN + lax.broadcasted_iota(jnp.int32, (BN, 1), 0)
    h2 = jnp.where(rows < N, h2, 0.0)
    blk = jnp.sum(h2, axis=0, keepdims=True)

    @pl.when(i == 0)
    def _():
        psum[...] = blk

    @pl.when(i > 0)
    def _():
        psum[...] = psum[...] + blk

    @pl.when(i == NGRID - 1)
    def _():
        g = psum[...] / float(N)
        logits = lax.dot_general(g, wfc[...], (((1,), (1,)), ((), ())),
                                 preferred_element_type=jnp.float32) + bfc[...]
        m = jnp.max(logits)
        e = jnp.exp(logits - m)
        out[...] = e / jnp.sum(e)


def _tc_layer2(accp, degf, h1, w2r, b2l, wfc, bfc):
    return pl.pallas_call(
        _tc2_body,
        grid=(NGRID,),
        in_specs=[
            pl.BlockSpec((1, BN, 32), lambda i: (0, i, 0)),
            pl.BlockSpec((1, BN, 32), lambda i: (1, i, 0)),
            pl.BlockSpec((BN,), lambda i: (i,)),
            pl.BlockSpec((BN,), lambda i: (NGRID + i,)),
            pl.BlockSpec((BN, 128), lambda i: (i, 0)),
            pl.BlockSpec((32, 128), lambda i: (0, 0)),
            pl.BlockSpec((1, 32), lambda i: (0, 0)),
            pl.BlockSpec((16, 32), lambda i: (0, 0)),
            pl.BlockSpec((1, 16), lambda i: (0, 0)),
        ],
        out_specs=pl.BlockSpec((1, 16), lambda i: (0, 0)),
        out_shape=jax.ShapeDtypeStruct((1, 16), jnp.float32),
        scratch_shapes=[pltpu.VMEM((1, 32), jnp.float32)],
    )(accp, accp, degf, degf, h1, w2r, b2l, wfc, bfc)


def kernel(x, edge_index, w1_l, b1_l, w1_r, w2_l, b2_l, w2_r, w_fc, b_fc):
    pad_iota = jnp.arange(EPAD - E, dtype=jnp.int32)
    srcp = jnp.concatenate([edge_index[0], pad_iota % N])
    dstp = jnp.concatenate([edge_index[1], N + pad_iota % (NP - N)])
    src1 = srcp.reshape(NW, NBLK1, IB1, CH)
    dst1 = dstp.reshape(NW, NBLK1, IB1, CH)
    src2 = srcp.reshape(NW, NBLK2, IB2, CH)
    dst2 = dstp.reshape(NW, NBLK2, IB2, CH)
    xp = jnp.pad(x, ((0, NP - N), (0, 0)))
    z128 = jnp.zeros((RPT, 128), jnp.float32)
    z32 = jnp.zeros((RPT, 32), jnp.float32)
    z16 = jnp.zeros((RPT, 16), jnp.float32)
    ones = jnp.ones((CH, 16), jnp.float32)

    acc1p, degf = _sc_aggregate(xp, src1, dst1, z128, True, z16, ones,
                                nblk=NBLK1, ib=IB1)
    acc1p = acc1p.reshape(NC, NP, 128)

    h1, y2 = _tc_layer1(acc1p, degf, x, w1_l, b1_l.reshape(1, 128), w1_r, w2_l)

    (acc2p,) = _sc_aggregate(y2, src2, dst2, z32, False, nblk=NBLK2, ib=IB2)
    acc2p = acc2p.reshape(NC, NP, 32)

    return _tc_layer2(acc2p, degf, h1, w2_r, b2_l.reshape(1, 32),
                      w_fc, b_fc.reshape(1, 16))
